# baseline (speedup 1.0000x reference)
import functools

import jax
import jax.numpy as jnp
from jax import lax
from jax.experimental import pallas as pl
from jax.experimental.pallas import tpu as pltpu
from jax.experimental.pallas import tpu_sc as plsc

_NUM_ACTIONS = 100000
_DIM = 64
_BATCH = 16384

_NC, _NS = 2, 16
_NW = _NC * _NS
_BPW = _BATCH // _NW
_LANES = 16
_WCH = 64


def _gather_body(actions_hbm, table_hbm, out_hbm, idx_v, rows_v, sem, wsem):
    wid = lax.axis_index("s") * _NC + lax.axis_index("c")
    base = wid * _BPW
    pltpu.sync_copy(actions_hbm.at[pl.ds(base, _BPW)], idx_v)

    def chunk(c, carry):
        vec = idx_v[pl.ds(c * _LANES, _LANES)]
        for l in range(_LANES):
            pltpu.async_copy(
                table_hbm.at[vec[l]], rows_v.at[c * _LANES + l], sem
            )
        return carry

    lax.fori_loop(0, _BPW // _LANES, chunk, 0)

    def wchunk(c, carry):
        pltpu.make_async_copy(
            table_hbm.at[pl.ds(0, _WCH)],
            rows_v.at[pl.ds(c * _WCH, _WCH)],
            sem,
        ).wait()
        pltpu.async_copy(
            rows_v.at[pl.ds(c * _WCH, _WCH)],
            out_hbm.at[pl.ds(base + c * _WCH, _WCH)],
            wsem,
        )
        return carry

    lax.fori_loop(0, _BPW // _WCH, wchunk, 0)
    pltpu.make_async_copy(
        rows_v, out_hbm.at[pl.ds(base, _BPW)], wsem
    ).wait()


def kernel(actions, embedding_weight):
    actions = actions.astype(jnp.int32)
    mesh = plsc.VectorSubcoreMesh(core_axis_name="c", subcore_axis_name="s")
    run = pl.kernel(
        _gather_body,
        mesh=mesh,
        out_type=jax.ShapeDtypeStruct((_BATCH, _DIM), jnp.float32),
        scratch_types=[
            pltpu.VMEM((_BPW,), jnp.int32),
            pltpu.VMEM((_BPW, _DIM), jnp.float32),
            pltpu.SemaphoreType.DMA,
            pltpu.SemaphoreType.DMA,
        ],
    )
    return run(actions, embedding_weight)


# speedup vs baseline: 1.0112x; 1.0112x over previous
import functools

import jax
import jax.numpy as jnp
from jax import lax
from jax.experimental import pallas as pl
from jax.experimental.pallas import tpu as pltpu
from jax.experimental.pallas import tpu_sc as plsc

_NUM_ACTIONS = 100000
_DIM = 64
_BATCH = 16384

_NC, _NS = 2, 16
_NW = _NC * _NS
_BPW = _BATCH // _NW
_LANES = 16


def _gather_body(actions_hbm, table_hbm, out_hbm, idx_v, rows_v, sem):
    wid = lax.axis_index("s") * _NC + lax.axis_index("c")
    base = wid * _BPW
    pltpu.sync_copy(actions_hbm.at[pl.ds(base, _BPW)], idx_v)

    def chunk(c, carry):
        vec = idx_v[pl.ds(c * _LANES, _LANES)]
        for l in range(_LANES):
            pltpu.async_copy(
                table_hbm.at[vec[l]], rows_v.at[c * _LANES + l], sem
            )
        return carry

    lax.fori_loop(0, _BPW // _LANES, chunk, 0)

    pltpu.make_async_copy(
        table_hbm.at[pl.ds(0, _BPW)], rows_v, sem
    ).wait()
    pltpu.sync_copy(rows_v, out_hbm.at[pl.ds(base, _BPW)])


def kernel(actions, embedding_weight):
    actions = actions.astype(jnp.int32)
    mesh = plsc.VectorSubcoreMesh(core_axis_name="c", subcore_axis_name="s")
    run = pl.kernel(
        _gather_body,
        mesh=mesh,
        out_type=jax.ShapeDtypeStruct((_BATCH, _DIM), jnp.float32),
        scratch_types=[
            pltpu.VMEM((_BPW,), jnp.int32),
            pltpu.VMEM((_BPW, _DIM), jnp.float32),
            pltpu.SemaphoreType.DMA,
        ],
    )
    return run(actions, embedding_weight)
